# Initial kernel scaffold; baseline (speedup 1.0000x reference)
import jax
import jax.numpy as jnp
from jax.experimental import pallas as pl


def kernel(d, t, n, c, edge_index, edge_type, W1, b1, weight, root, bias):
    raise NotImplementedError("write your pallas kernel here")


# speedup vs baseline: 4.3590x; 4.3590x over previous
import functools

import jax
import jax.numpy as jnp
from jax import lax
from jax.experimental import pallas as pl
from jax.experimental.pallas import tpu as pltpu
from jax.experimental.pallas import tpu_sc as plsc

N = 10000
D = 128
NUM_REL = 2
NC = 2
NS = 16
L = 16

CHUNK = 128
N_PAD = 10240
ROWS_PER_TILE = N_PAD // NS
E_PAD = 16 * CHUNK * 157
EPT = E_PAD // NS
NCHUNKS = EPT // CHUNK


def _sc_agg_body(x_hbm, src_hbm, dst_hbm, et_hbm, out_hbm,
                 agg_sh, src_v, dst_v, et_v, sidx_v, rows_v, sem):
    c = lax.axis_index("c")
    s = lax.axis_index("s")
    row0 = s * ROWS_PER_TILE

    zero16 = jnp.zeros((L,), jnp.float32)

    def zrow(i, carry):
        for j in range(D // L):
            rows_v[0, i, pl.ds(j * L, L)] = zero16
        return carry

    lax.fori_loop(0, CHUNK, zrow, 0)
    for k in range(ROWS_PER_TILE // CHUNK):
        pltpu.sync_copy(rows_v.at[0], agg_sh.at[pl.ds(row0 + k * CHUNK, CHUNK)])
    plsc.subcore_barrier()

    iota16 = lax.iota(jnp.int32, L)
    dump_vec = N + s * 8 + (iota16 & 7)
    ebase = s * EPT

    def step(i, carry):
        off = ebase + i * CHUNK
        pltpu.sync_copy(src_hbm.at[pl.ds(off, CHUNK)], src_v.at[0])
        pltpu.sync_copy(dst_hbm.at[pl.ds(off, CHUNK)], dst_v.at[0])
        pltpu.sync_copy(et_hbm.at[pl.ds(off, CHUNK)], et_v.at[0])
        for j in range(CHUNK // L):
            t16 = et_v[0, pl.ds(j * L, L)]
            d16 = dst_v[0, pl.ds(j * L, L)]
            sidx_v[0, pl.ds(j * L, L)] = jnp.where(t16 == c, d16, dump_vec)
        pltpu.async_copy(x_hbm.at[src_v.at[0]], rows_v.at[0], sem).wait()
        pltpu.sync_copy(rows_v.at[0], agg_sh.at[sidx_v.at[0]], add=True)
        return carry

    lax.fori_loop(0, NCHUNKS, step, 0)
    plsc.subcore_barrier()
    pltpu.sync_copy(agg_sh.at[pl.ds(row0, ROWS_PER_TILE)],
                    out_hbm.at[c, pl.ds(row0, ROWS_PER_TILE)])


def _sc_counts_body(dst_hbm, et_hbm, out_hbm,
                    cnt_sh, dst_v, et_v, sidx_v, val_v):
    c = lax.axis_index("c")
    s = lax.axis_index("s")
    row0 = s * ROWS_PER_TILE
    zero16 = jnp.zeros((L,), jnp.float32)
    one16 = jnp.ones((L,), jnp.float32)

    def fill(vec):
        def frow(i, carry):
            for j in range(D // L):
                val_v[i, pl.ds(j * L, L)] = vec
            return carry
        lax.fori_loop(0, CHUNK, frow, 0)

    fill(zero16)
    for k in range(ROWS_PER_TILE // CHUNK):
        pltpu.sync_copy(val_v, cnt_sh.at[pl.ds(row0 + k * CHUNK, CHUNK)])
    plsc.subcore_barrier()
    fill(one16)

    iota16 = lax.iota(jnp.int32, L)
    dump_vec = N + s * 8 + (iota16 & 7)
    ebase = s * EPT

    def step(i, carry):
        off = ebase + i * CHUNK
        pltpu.sync_copy(dst_hbm.at[pl.ds(off, CHUNK)], dst_v.at[0])
        pltpu.sync_copy(et_hbm.at[pl.ds(off, CHUNK)], et_v.at[0])
        for j in range(CHUNK // L):
            t16 = et_v[0, pl.ds(j * L, L)]
            d16 = dst_v[0, pl.ds(j * L, L)]
            sidx_v[0, pl.ds(j * L, L)] = jnp.where(t16 == c, d16, dump_vec)
        pltpu.sync_copy(val_v, cnt_sh.at[sidx_v.at[0]], add=True)
        return carry

    lax.fori_loop(0, NCHUNKS, step, 0)
    plsc.subcore_barrier()
    pltpu.sync_copy(cnt_sh.at[pl.ds(row0, ROWS_PER_TILE)],
                    out_hbm.at[c, pl.ds(row0, ROWS_PER_TILE)])


def _sc_mesh():
    return plsc.VectorSubcoreMesh(core_axis_name="c", subcore_axis_name="s",
                                  num_cores=NC, num_subcores=NS)


def _sc_agg(x, src_p, dst_p, et_p):
    return pl.kernel(
        _sc_agg_body,
        out_type=jax.ShapeDtypeStruct((NUM_REL, N_PAD, D), jnp.float32),
        mesh=_sc_mesh(),
        scratch_types=[
            pltpu.VMEM_SHARED((N_PAD, D), jnp.float32),
            pltpu.VMEM((1, CHUNK), jnp.int32),
            pltpu.VMEM((1, CHUNK), jnp.int32),
            pltpu.VMEM((1, CHUNK), jnp.int32),
            pltpu.VMEM((1, CHUNK), jnp.int32),
            pltpu.VMEM((1, CHUNK, D), jnp.float32),
            pltpu.SemaphoreType.DMA,
        ],
    )(x, src_p, dst_p, et_p)


def _sc_counts(dst_p, et_p):
    return pl.kernel(
        _sc_counts_body,
        out_type=jax.ShapeDtypeStruct((NUM_REL, N_PAD, D), jnp.float32),
        mesh=_sc_mesh(),
        scratch_types=[
            pltpu.VMEM_SHARED((N_PAD, D), jnp.float32),
            pltpu.VMEM((1, CHUNK), jnp.int32),
            pltpu.VMEM((1, CHUNK), jnp.int32),
            pltpu.VMEM((1, CHUNK), jnp.int32),
            pltpu.VMEM((CHUNK, D), jnp.float32),
        ],
    )(dst_p, et_p)


BN = 1000


def _tc_in_body(x0_ref, w_ref, b_ref, o_ref):
    h = lax.dot_general(x0_ref[...], w_ref[...], (((1,), (1,)), ((), ())),
                        preferred_element_type=jnp.float32) + b_ref[...]
    o_ref[...] = jnp.where(h >= 0, h, 0.01 * h)


def _tc_in(x0, W1, b1):
    return pl.pallas_call(
        _tc_in_body,
        grid=(N // BN,),
        in_specs=[
            pl.BlockSpec((BN, D), lambda i: (i, 0)),
            pl.BlockSpec((D, D), lambda i: (0, 0)),
            pl.BlockSpec((1, D), lambda i: (0, 0)),
        ],
        out_specs=pl.BlockSpec((BN, D), lambda i: (i, 0)),
        out_shape=jax.ShapeDtypeStruct((N, D), jnp.float32),
    )(x0, W1, b1.reshape(1, D))


def _tc_combine_body(x_ref, a0_ref, a1_ref, c0_ref, c1_ref,
                     root_ref, w0_ref, w1_ref, b_ref, o_ref):
    inv0 = 1.0 / jnp.maximum(c0_ref[0, :, 0:1], 1.0)
    inv1 = 1.0 / jnp.maximum(c1_ref[0, :, 0:1], 1.0)
    acc = jnp.dot(x_ref[...], root_ref[...], preferred_element_type=jnp.float32)
    acc = acc + jnp.dot(a0_ref[0] * inv0, w0_ref[...],
                        preferred_element_type=jnp.float32)
    acc = acc + jnp.dot(a1_ref[0] * inv1, w1_ref[...],
                        preferred_element_type=jnp.float32)
    o_ref[...] = acc + b_ref[...]


def _tc_combine(x, agg, cnt, root, weight, bias):
    return pl.pallas_call(
        _tc_combine_body,
        grid=(N // BN,),
        in_specs=[
            pl.BlockSpec((BN, D), lambda i: (i, 0)),
            pl.BlockSpec((1, BN, D), lambda i: (0, i, 0)),
            pl.BlockSpec((1, BN, D), lambda i: (1, i, 0)),
            pl.BlockSpec((1, BN, D), lambda i: (0, i, 0)),
            pl.BlockSpec((1, BN, D), lambda i: (1, i, 0)),
            pl.BlockSpec((D, D), lambda i: (0, 0)),
            pl.BlockSpec((D, D), lambda i: (0, 0)),
            pl.BlockSpec((D, D), lambda i: (0, 0)),
            pl.BlockSpec((1, D), lambda i: (0, 0)),
        ],
        out_specs=pl.BlockSpec((BN, D), lambda i: (i, 0)),
        out_shape=jax.ShapeDtypeStruct((N, D), jnp.float32),
    )(x, agg, agg, cnt, cnt, root, weight[0], weight[1], bias.reshape(1, D))


def kernel(d, t, n, c, edge_index, edge_type, W1, b1, weight, root, bias):
    x0 = jnp.concatenate((d, t, n, c), axis=1)
    pad = E_PAD - edge_index.shape[1]
    src_p = jnp.concatenate([edge_index[0], jnp.zeros((pad,), jnp.int32)])
    dst_p = jnp.concatenate([edge_index[1], jnp.zeros((pad,), jnp.int32)])
    et_p = jnp.concatenate([edge_type, jnp.full((pad,), NUM_REL, jnp.int32)])

    x = _tc_in(x0, W1, b1)
    cnt = _sc_counts(dst_p, et_p)
    agg1 = _sc_agg(x, src_p, dst_p, et_p)
    h = _tc_combine(x, agg1, cnt, root, weight, bias)
    agg2 = _sc_agg(h, src_p, dst_p, et_p)
    return _tc_combine(h, agg2, cnt, root, weight, bias)
